# baseline (speedup 1.0000x reference)
import functools

import jax
import jax.numpy as jnp
from jax import lax
from jax.experimental import pallas as pl

EPS_ = 1e-06
H_ = 4
DK_ = 32
D_ = 128
CAT_ = 256

EB_ = 1000
NT_ = 1000
NB_ = 1000

F32 = jnp.float32


def _sel_cols(off):
    r = jnp.arange(CAT_)[:, None]
    c = jnp.arange(D_)[None, :]
    return ((r - off == c) & (c < H_)).astype(F32)


def _sel_z():
    r = jnp.arange(CAT_)[:, None]
    c = jnp.arange(D_)[None, :]
    return (r == c).astype(F32)


def _sel_denom():
    r = jnp.arange(CAT_)[:, None]
    c = jnp.arange(D_)[None, :]
    return (r - D_ == c // DK_).astype(F32)


def _spread_heads():
    r = jnp.arange(D_)[:, None]
    c = jnp.arange(D_)[None, :]
    return (r == c // DK_).astype(F32)


def _keep4():
    r = jnp.arange(D_)[:, None]
    c = jnp.arange(D_)[None, :]
    return ((r == c) & (c < H_)).astype(F32)


def _linear_body(h_ref, wt_ref, b_ref, a_ref, out_ref):
    lin = jnp.dot(h_ref[...], wt_ref[...], preferred_element_type=F32) + b_ref[...]
    proj = jnp.dot(lin, a_ref[...], preferred_element_type=F32)
    out_ref[...] = jnp.concatenate([lin, proj], axis=1)


def _linear(h, wt, b, a):
    n = h.shape[0]
    grid = (n // NB_,)
    return pl.pallas_call(
        _linear_body,
        grid=grid,
        in_specs=[
            pl.BlockSpec((NB_, D_), lambda i: (i, 0)),
            pl.BlockSpec((D_, D_), lambda i: (0, 0)),
            pl.BlockSpec((1, D_), lambda i: (0, 0)),
            pl.BlockSpec((D_, D_), lambda i: (0, 0)),
        ],
        out_specs=pl.BlockSpec((NB_, CAT_), lambda i: (i, 0)),
        out_shape=jax.ShapeDtypeStruct((n, CAT_), F32),
    )(h, wt, b, a)


def _gather_body(idx_ref, tab_ref, out_ref):
    j = pl.program_id(1)

    @pl.when(j == 0)
    def _():
        out_ref[...] = jnp.zeros_like(out_ref)

    ids = idx_ref[0]
    rowid = lax.broadcasted_iota(jnp.int32, (NT_, 1), 0) + j * NT_
    oh = (rowid == ids).astype(F32)
    out_ref[...] += lax.dot_general(
        oh, tab_ref[...], (((0,), (0,)), ((), ())), preferred_element_type=F32)


def _gather(tab, idx):
    e = idx.shape[0]
    n = tab.shape[0]
    idx3 = idx.reshape(e // EB_, 1, EB_)
    grid = (e // EB_, n // NT_)
    return pl.pallas_call(
        _gather_body,
        grid=grid,
        in_specs=[
            pl.BlockSpec((1, 1, EB_), lambda i, j: (i, 0, 0)),
            pl.BlockSpec((NT_, CAT_), lambda i, j: (j, 0)),
        ],
        out_specs=pl.BlockSpec((EB_, CAT_), lambda i, j: (i, 0)),
        out_shape=jax.ShapeDtypeStruct((e, CAT_), F32),
    )(idx3, tab)


def _scatter_body(idx_ref, val_ref, out_ref):
    i = pl.program_id(0)
    j = pl.program_id(1)

    @pl.when(j == 0)
    def _():
        out_ref[...] = jnp.zeros_like(out_ref)

    ids = idx_ref[0]
    rowid = lax.broadcasted_iota(jnp.int32, (NT_, 1), 0) + i * NT_
    oh = (rowid == ids).astype(F32)
    out_ref[...] += jnp.dot(oh, val_ref[...], preferred_element_type=F32)


def _scatter(vals, idx, n):
    e = idx.shape[0]
    idx3 = idx.reshape(e // EB_, 1, EB_)
    grid = (n // NT_, e // EB_)
    return pl.pallas_call(
        _scatter_body,
        grid=grid,
        in_specs=[
            pl.BlockSpec((1, 1, EB_), lambda i, j: (j, 0, 0)),
            pl.BlockSpec((EB_, CAT_), lambda i, j: (j, 0)),
        ],
        out_specs=pl.BlockSpec((NT_, CAT_), lambda i, j: (i, 0)),
        out_shape=jax.ShapeDtypeStruct((n, CAT_), F32),
    )(idx3, vals)


def _edge_body(gs_ref, gd_ref, feat_ref, selp_ref, selpd_ref, attf_ref,
               selz_ref, sprd_ref, keep_ref, out_ref):
    gs = gs_ref[...]
    gd = gd_ref[...]
    zs = jnp.dot(gs, selz_ref[...], preferred_element_type=F32)
    s = (jnp.dot(gs, selp_ref[...], preferred_element_type=F32)
         + jnp.dot(gd, selpd_ref[...], preferred_element_type=F32)
         + feat_ref[...] * attf_ref[...])
    s = jnp.where(s >= 0, s, 0.2 * s)
    al = jnp.exp(s)
    w = zs * jnp.dot(al, sprd_ref[...], preferred_element_type=F32)
    alk = jnp.dot(al, keep_ref[...], preferred_element_type=F32)
    out_ref[...] = jnp.concatenate([w, alk], axis=1)


def _edge(gs, gd, feat, selp, selpd, attf):
    e = gs.shape[0]
    grid = (e // EB_,)
    return pl.pallas_call(
        _edge_body,
        grid=grid,
        in_specs=[
            pl.BlockSpec((EB_, CAT_), lambda i: (i, 0)),
            pl.BlockSpec((EB_, CAT_), lambda i: (i, 0)),
            pl.BlockSpec((EB_, 1), lambda i: (i, 0)),
            pl.BlockSpec((CAT_, D_), lambda i: (0, 0)),
            pl.BlockSpec((CAT_, D_), lambda i: (0, 0)),
            pl.BlockSpec((1, D_), lambda i: (0, 0)),
            pl.BlockSpec((CAT_, D_), lambda i: (0, 0)),
            pl.BlockSpec((D_, D_), lambda i: (0, 0)),
            pl.BlockSpec((D_, D_), lambda i: (0, 0)),
        ],
        out_specs=pl.BlockSpec((EB_, CAT_), lambda i: (i, 0)),
        out_shape=jax.ShapeDtypeStruct((e, CAT_), F32),
    )(gs, gd, feat, selp, selpd, attf, _sel_z(), _spread_heads(), _keep4())


def _final2_body(a1_ref, a2_ref, lin_ref, selz_ref, seld_ref, out_ref):
    selz = selz_ref[...]
    seld = seld_ref[...]
    a1 = a1_ref[...]
    a2 = a2_ref[...]
    n1 = jnp.dot(a1, selz, preferred_element_type=F32) / (
        jnp.dot(a1, seld, preferred_element_type=F32) + EPS_)
    n2 = jnp.dot(a2, selz, preferred_element_type=F32) / (
        jnp.dot(a2, seld, preferred_element_type=F32) + EPS_)
    z = jnp.dot(lin_ref[...], selz, preferred_element_type=F32)
    x = n1 + n2 + z
    out_ref[...] = jnp.where(x > 0, x, jnp.exp(jnp.minimum(x, 0.0)) - 1.0)


def _final1_body(a1_ref, lin_ref, selz_ref, seld_ref, out_ref):
    selz = selz_ref[...]
    seld = seld_ref[...]
    a1 = a1_ref[...]
    n1 = jnp.dot(a1, selz, preferred_element_type=F32) / (
        jnp.dot(a1, seld, preferred_element_type=F32) + EPS_)
    z = jnp.dot(lin_ref[...], selz, preferred_element_type=F32)
    x = n1 + z
    out_ref[...] = jnp.where(x > 0, x, jnp.exp(jnp.minimum(x, 0.0)) - 1.0)


def _final(aggs, lin_cat):
    n = lin_cat.shape[0]
    grid = (n // NB_,)
    body = _final2_body if len(aggs) == 2 else _final1_body
    num_big = len(aggs) + 1
    return pl.pallas_call(
        body,
        grid=grid,
        in_specs=(
            [pl.BlockSpec((NB_, CAT_), lambda i: (i, 0)) for _ in range(num_big)]
            + [pl.BlockSpec((CAT_, D_), lambda i: (0, 0)),
               pl.BlockSpec((CAT_, D_), lambda i: (0, 0))]
        ),
        out_specs=pl.BlockSpec((NB_, D_), lambda i: (i, 0)),
        out_shape=jax.ShapeDtypeStruct((n, D_), F32),
    )(*aggs, lin_cat, _sel_z(), _sel_denom())


def _att_matrix(att_list):
    a = jnp.zeros((D_, D_), dtype=F32)
    for slot, (att, col0) in enumerate(att_list):
        vals = att[:, col0:col0 + DK_].reshape(-1)
        rows = jnp.arange(D_)
        cols = H_ * slot + rows // DK_
        a = a.at[rows, cols].add(vals)
    return a


def _attf_row(att):
    row = jnp.zeros((D_,), dtype=F32).at[jnp.arange(H_)].set(att[:, 2 * DK_])
    return row.reshape(1, D_)


def kernel(h_op, h_mac, seq_src, seq_dst, seq_feat, op_mac_src, op_mac_dst,
           op_mac_feat, mac_op_src, mac_op_dst, mac_op_feat, W_op, b_op,
           W_mac, b_mac, att_seq, att_op_mac, att_mac_op):
    n_op = h_op.shape[0]
    n_mac = h_mac.shape[0]

    a_op = _att_matrix([(att_seq, 0), (att_seq, DK_), (att_op_mac, 0),
                        (att_mac_op, DK_)])
    a_mac = _att_matrix([(att_op_mac, DK_), (att_mac_op, 0)])

    t_op = _linear(h_op, W_op.T, b_op.reshape(1, D_), a_op)
    t_mac = _linear(h_mac, W_mac.T, b_mac.reshape(1, D_), a_mac)

    def relation(t_src, t_dst, src, dst, feat, off_s, off_d, att, n_dst):
        gs = _gather(t_src, src)
        gd = _gather(t_dst, dst)
        cat = _edge(gs, gd, feat, _sel_cols(D_ + off_s), _sel_cols(D_ + off_d),
                    _attf_row(att))
        return _scatter(cat, dst, n_dst)

    agg_seq = relation(t_op, t_op, seq_src, seq_dst, seq_feat, 0, 4,
                       att_seq, n_op)
    agg_opmac = relation(t_op, t_mac, op_mac_src, op_mac_dst, op_mac_feat,
                         8, 0, att_op_mac, n_mac)
    agg_macop = relation(t_mac, t_op, mac_op_src, mac_op_dst, mac_op_feat,
                         4, 12, att_mac_op, n_op)

    res_op = _final([agg_seq, agg_macop], t_op)
    res_mac = _final([agg_opmac], t_mac)
    return (res_op, res_mac)


# speedup vs baseline: 1.0858x; 1.0858x over previous
import functools

import jax
import jax.numpy as jnp
from jax import lax
from jax.experimental import pallas as pl

EPS_ = 1e-06
H_ = 4
DK_ = 32
D_ = 128
CAT_ = 256

EB_ = 1000
NT_ = 1000
NB_ = 1000

F32 = jnp.float32


def _sel_cols(off, in_dim=CAT_):
    r = jnp.arange(in_dim)[:, None]
    c = jnp.arange(D_)[None, :]
    return ((r - off == c) & (c < H_)).astype(F32)


def _sel_z():
    r = jnp.arange(CAT_)[:, None]
    c = jnp.arange(D_)[None, :]
    return (r == c).astype(F32)


def _sel_denom():
    r = jnp.arange(CAT_)[:, None]
    c = jnp.arange(D_)[None, :]
    return (r - D_ == c // DK_).astype(F32)


def _spread_heads():
    r = jnp.arange(D_)[:, None]
    c = jnp.arange(D_)[None, :]
    return (r == c // DK_).astype(F32)


def _keep4():
    r = jnp.arange(D_)[:, None]
    c = jnp.arange(D_)[None, :]
    return ((r == c) & (c < H_)).astype(F32)


def _linear_body(h_ref, wt_ref, b_ref, a_ref, out_ref):
    lin = jnp.dot(h_ref[...], wt_ref[...], preferred_element_type=F32) + b_ref[...]
    proj = jnp.dot(lin, a_ref[...], preferred_element_type=F32)
    out_ref[...] = jnp.concatenate([lin, proj], axis=1)


def _linear(h, wt, b, a):
    n = h.shape[0]
    grid = (n // NB_,)
    return pl.pallas_call(
        _linear_body,
        grid=grid,
        in_specs=[
            pl.BlockSpec((NB_, D_), lambda i: (i, 0)),
            pl.BlockSpec((D_, D_), lambda i: (0, 0)),
            pl.BlockSpec((1, D_), lambda i: (0, 0)),
            pl.BlockSpec((D_, D_), lambda i: (0, 0)),
        ],
        out_specs=pl.BlockSpec((NB_, CAT_), lambda i: (i, 0)),
        out_shape=jax.ShapeDtypeStruct((n, CAT_), F32),
    )(h, wt, b, a)


def _gather_body(idx_ref, tab_ref, out_ref):
    j = pl.program_id(1)

    @pl.when(j == 0)
    def _():
        out_ref[...] = jnp.zeros_like(out_ref)

    ids = idx_ref[0]
    rowid = lax.broadcasted_iota(jnp.int32, (NT_, 1), 0) + j * NT_
    oh = (rowid == ids).astype(jnp.bfloat16)
    out_ref[...] += lax.dot_general(
        oh, tab_ref[...].astype(jnp.bfloat16), (((0,), (0,)), ((), ())),
        preferred_element_type=F32)


def _gather(tab, idx):
    e = idx.shape[0]
    n, c = tab.shape
    idx3 = idx.reshape(e // EB_, 1, EB_)
    grid = (e // EB_, n // NT_)
    return pl.pallas_call(
        _gather_body,
        grid=grid,
        in_specs=[
            pl.BlockSpec((1, 1, EB_), lambda i, j: (i, 0, 0)),
            pl.BlockSpec((NT_, c), lambda i, j: (j, 0)),
        ],
        out_specs=pl.BlockSpec((EB_, c), lambda i, j: (i, 0)),
        out_shape=jax.ShapeDtypeStruct((e, c), F32),
    )(idx3, tab)


def _scatter_body(idx_ref, val_ref, out_ref):
    i = pl.program_id(0)
    j = pl.program_id(1)

    @pl.when(j == 0)
    def _():
        out_ref[...] = jnp.zeros_like(out_ref)

    ids = idx_ref[0]
    rowid = lax.broadcasted_iota(jnp.int32, (NT_, 1), 0) + i * NT_
    oh = (rowid == ids).astype(jnp.bfloat16)
    out_ref[...] += jnp.dot(oh, val_ref[...].astype(jnp.bfloat16),
                            preferred_element_type=F32)


def _scatter(vals, idx, n):
    e = idx.shape[0]
    idx3 = idx.reshape(e // EB_, 1, EB_)
    grid = (n // NT_, e // EB_)
    return pl.pallas_call(
        _scatter_body,
        grid=grid,
        in_specs=[
            pl.BlockSpec((1, 1, EB_), lambda i, j: (j, 0, 0)),
            pl.BlockSpec((EB_, CAT_), lambda i, j: (j, 0)),
        ],
        out_specs=pl.BlockSpec((NT_, CAT_), lambda i, j: (i, 0)),
        out_shape=jax.ShapeDtypeStruct((n, CAT_), F32),
    )(idx3, vals)


def _edge_body(gs_ref, gd_ref, feat_ref, selp_ref, selpd_ref, attf_ref,
               selz_ref, sprd_ref, keep_ref, out_ref):
    gs = gs_ref[...]
    gd = gd_ref[...]
    zs = jnp.dot(gs, selz_ref[...], preferred_element_type=F32)
    s = (jnp.dot(gs, selp_ref[...], preferred_element_type=F32)
         + jnp.dot(gd, selpd_ref[...], preferred_element_type=F32)
         + feat_ref[...] * attf_ref[...])
    s = jnp.where(s >= 0, s, 0.2 * s)
    al = jnp.exp(s)
    w = zs * jnp.dot(al, sprd_ref[...], preferred_element_type=F32)
    alk = jnp.dot(al, keep_ref[...], preferred_element_type=F32)
    out_ref[...] = jnp.concatenate([w, alk], axis=1)


def _edge(gs, gd, feat, selp, selpd, attf):
    e = gs.shape[0]
    cd = gd.shape[1]
    grid = (e // EB_,)
    return pl.pallas_call(
        _edge_body,
        grid=grid,
        in_specs=[
            pl.BlockSpec((EB_, CAT_), lambda i: (i, 0)),
            pl.BlockSpec((EB_, cd), lambda i: (i, 0)),
            pl.BlockSpec((EB_, 1), lambda i: (i, 0)),
            pl.BlockSpec((CAT_, D_), lambda i: (0, 0)),
            pl.BlockSpec((cd, D_), lambda i: (0, 0)),
            pl.BlockSpec((1, D_), lambda i: (0, 0)),
            pl.BlockSpec((CAT_, D_), lambda i: (0, 0)),
            pl.BlockSpec((D_, D_), lambda i: (0, 0)),
            pl.BlockSpec((D_, D_), lambda i: (0, 0)),
        ],
        out_specs=pl.BlockSpec((EB_, CAT_), lambda i: (i, 0)),
        out_shape=jax.ShapeDtypeStruct((e, CAT_), F32),
    )(gs, gd, feat, selp, selpd, attf, _sel_z(), _spread_heads(), _keep4())


def _final2_body(a1_ref, a2_ref, lin_ref, selz_ref, seld_ref, out_ref):
    selz = selz_ref[...]
    seld = seld_ref[...]
    a1 = a1_ref[...]
    a2 = a2_ref[...]
    n1 = jnp.dot(a1, selz, preferred_element_type=F32) / (
        jnp.dot(a1, seld, preferred_element_type=F32) + EPS_)
    n2 = jnp.dot(a2, selz, preferred_element_type=F32) / (
        jnp.dot(a2, seld, preferred_element_type=F32) + EPS_)
    z = jnp.dot(lin_ref[...], selz, preferred_element_type=F32)
    x = n1 + n2 + z
    out_ref[...] = jnp.where(x > 0, x, jnp.exp(jnp.minimum(x, 0.0)) - 1.0)


def _final1_body(a1_ref, lin_ref, selz_ref, seld_ref, out_ref):
    selz = selz_ref[...]
    seld = seld_ref[...]
    a1 = a1_ref[...]
    n1 = jnp.dot(a1, selz, preferred_element_type=F32) / (
        jnp.dot(a1, seld, preferred_element_type=F32) + EPS_)
    z = jnp.dot(lin_ref[...], selz, preferred_element_type=F32)
    x = n1 + z
    out_ref[...] = jnp.where(x > 0, x, jnp.exp(jnp.minimum(x, 0.0)) - 1.0)


def _final(aggs, lin_cat):
    n = lin_cat.shape[0]
    grid = (n // NB_,)
    body = _final2_body if len(aggs) == 2 else _final1_body
    num_big = len(aggs) + 1
    return pl.pallas_call(
        body,
        grid=grid,
        in_specs=(
            [pl.BlockSpec((NB_, CAT_), lambda i: (i, 0)) for _ in range(num_big)]
            + [pl.BlockSpec((CAT_, D_), lambda i: (0, 0)),
               pl.BlockSpec((CAT_, D_), lambda i: (0, 0))]
        ),
        out_specs=pl.BlockSpec((NB_, D_), lambda i: (i, 0)),
        out_shape=jax.ShapeDtypeStruct((n, D_), F32),
    )(*aggs, lin_cat, _sel_z(), _sel_denom())


def _att_matrix(att_list):
    a = jnp.zeros((D_, D_), dtype=F32)
    for slot, (att, col0) in enumerate(att_list):
        vals = att[:, col0:col0 + DK_].reshape(-1)
        rows = jnp.arange(D_)
        cols = H_ * slot + rows // DK_
        a = a.at[rows, cols].add(vals)
    return a


def _attf_row(att):
    row = jnp.zeros((D_,), dtype=F32).at[jnp.arange(H_)].set(att[:, 2 * DK_])
    return row.reshape(1, D_)


def kernel(h_op, h_mac, seq_src, seq_dst, seq_feat, op_mac_src, op_mac_dst,
           op_mac_feat, mac_op_src, mac_op_dst, mac_op_feat, W_op, b_op,
           W_mac, b_mac, att_seq, att_op_mac, att_mac_op):
    n_op = h_op.shape[0]
    n_mac = h_mac.shape[0]

    a_op = _att_matrix([(att_seq, 0), (att_seq, DK_), (att_op_mac, 0),
                        (att_mac_op, DK_)])
    a_mac = _att_matrix([(att_op_mac, DK_), (att_mac_op, 0)])

    t_op = _linear(h_op, W_op.T, b_op.reshape(1, D_), a_op)
    t_mac = _linear(h_mac, W_mac.T, b_mac.reshape(1, D_), a_mac)

    p_op = t_op[:, D_:]
    p_mac = t_mac[:, D_:]

    def relation(t_src, p_dst, src, dst, feat, off_s, off_d, att, n_dst):
        gs = _gather(t_src, src)
        gd = _gather(p_dst, dst)
        cat = _edge(gs, gd, feat, _sel_cols(D_ + off_s), _sel_cols(off_d, D_),
                    _attf_row(att))
        return _scatter(cat, dst, n_dst)

    agg_seq = relation(t_op, p_op, seq_src, seq_dst, seq_feat, 0, 4,
                       att_seq, n_op)
    agg_opmac = relation(t_op, p_mac, op_mac_src, op_mac_dst, op_mac_feat,
                         8, 0, att_op_mac, n_mac)
    agg_macop = relation(t_mac, p_op, mac_op_src, mac_op_dst, mac_op_feat,
                         4, 12, att_mac_op, n_op)

    res_op = _final([agg_seq, agg_macop], t_op)
    res_mac = _final([agg_opmac], t_mac)
    return (res_op, res_mac)


# speedup vs baseline: 1.6321x; 1.5031x over previous
import functools

import jax
import jax.numpy as jnp
from jax import lax
from jax.experimental import pallas as pl

EPS_ = 1e-06
H_ = 4
DK_ = 32
D_ = 128
CAT_ = 256

EB_ = 3000
NT_ = 1000
NB_ = 1000

F32 = jnp.float32


def _sel_cols(off, in_dim=CAT_):
    r = jnp.arange(in_dim)[:, None]
    c = jnp.arange(D_)[None, :]
    return ((r - off == c) & (c < H_)).astype(F32)


def _sel_z():
    r = jnp.arange(CAT_)[:, None]
    c = jnp.arange(D_)[None, :]
    return (r == c).astype(F32)


def _sel_denom():
    r = jnp.arange(CAT_)[:, None]
    c = jnp.arange(D_)[None, :]
    return (r - D_ == c // DK_).astype(F32)


def _spread_heads():
    r = jnp.arange(D_)[:, None]
    c = jnp.arange(D_)[None, :]
    return (r == c // DK_).astype(F32)


def _keep4():
    r = jnp.arange(D_)[:, None]
    c = jnp.arange(D_)[None, :]
    return ((r == c) & (c < H_)).astype(F32)


def _linear_body(h_ref, wt_ref, b_ref, a_ref, out_ref):
    lin = jnp.dot(h_ref[...], wt_ref[...], preferred_element_type=F32) + b_ref[...]
    proj = jnp.dot(lin, a_ref[...], preferred_element_type=F32)
    out_ref[...] = jnp.concatenate([lin, proj], axis=1)


def _linear(h, wt, b, a):
    n = h.shape[0]
    grid = (n // NB_,)
    return pl.pallas_call(
        _linear_body,
        grid=grid,
        in_specs=[
            pl.BlockSpec((NB_, D_), lambda i: (i, 0)),
            pl.BlockSpec((D_, D_), lambda i: (0, 0)),
            pl.BlockSpec((1, D_), lambda i: (0, 0)),
            pl.BlockSpec((D_, D_), lambda i: (0, 0)),
        ],
        out_specs=pl.BlockSpec((NB_, CAT_), lambda i: (i, 0)),
        out_shape=jax.ShapeDtypeStruct((n, CAT_), F32),
    )(h, wt, b, a)


def _gather_body(idx_ref, tab_ref, out_ref):
    j = pl.program_id(1)

    @pl.when(j == 0)
    def _():
        out_ref[...] = jnp.zeros_like(out_ref)

    ids = idx_ref[0]
    rowid = lax.broadcasted_iota(jnp.int32, (NT_, 1), 0) + j * NT_
    oh = (rowid == ids).astype(jnp.bfloat16)
    out_ref[...] += lax.dot_general(
        oh, tab_ref[...].astype(jnp.bfloat16), (((0,), (0,)), ((), ())),
        preferred_element_type=F32)


def _gather(tab, idx):
    e = idx.shape[0]
    n, c = tab.shape
    idx3 = idx.reshape(e // EB_, 1, EB_)
    grid = (e // EB_, n // NT_)
    return pl.pallas_call(
        _gather_body,
        grid=grid,
        in_specs=[
            pl.BlockSpec((1, 1, EB_), lambda i, j: (i, 0, 0)),
            pl.BlockSpec((NT_, c), lambda i, j: (j, 0)),
        ],
        out_specs=pl.BlockSpec((EB_, c), lambda i, j: (i, 0)),
        out_shape=jax.ShapeDtypeStruct((e, c), F32),
    )(idx3, tab)


def _scatter_body(idx_ref, val_ref, out_ref):
    i = pl.program_id(0)
    j = pl.program_id(1)

    @pl.when(j == 0)
    def _():
        out_ref[...] = jnp.zeros_like(out_ref)

    ids = idx_ref[0]
    rowid = lax.broadcasted_iota(jnp.int32, (NT_, 1), 0) + i * NT_
    oh = (rowid == ids).astype(jnp.bfloat16)
    out_ref[...] += jnp.dot(oh, val_ref[...].astype(jnp.bfloat16),
                            preferred_element_type=F32)


def _scatter(vals, idx, n):
    e = idx.shape[0]
    idx3 = idx.reshape(e // EB_, 1, EB_)
    grid = (n // NT_, e // EB_)
    return pl.pallas_call(
        _scatter_body,
        grid=grid,
        in_specs=[
            pl.BlockSpec((1, 1, EB_), lambda i, j: (j, 0, 0)),
            pl.BlockSpec((EB_, CAT_), lambda i, j: (j, 0)),
        ],
        out_specs=pl.BlockSpec((NT_, CAT_), lambda i, j: (i, 0)),
        out_shape=jax.ShapeDtypeStruct((n, CAT_), F32),
    )(idx3, vals)


def _edge_body(gs_ref, gd_ref, feat_ref, selp_ref, selpd_ref, attf_ref,
               selz_ref, sprd_ref, keep_ref, out_ref):
    gs = gs_ref[...]
    gd = gd_ref[...]
    zs = jnp.dot(gs, selz_ref[...], preferred_element_type=F32)
    s = (jnp.dot(gs, selp_ref[...], preferred_element_type=F32)
         + jnp.dot(gd, selpd_ref[...], preferred_element_type=F32)
         + feat_ref[...] * attf_ref[...])
    s = jnp.where(s >= 0, s, 0.2 * s)
    al = jnp.exp(s)
    w = zs * jnp.dot(al, sprd_ref[...], preferred_element_type=F32)
    alk = jnp.dot(al, keep_ref[...], preferred_element_type=F32)
    out_ref[...] = jnp.concatenate([w, alk], axis=1)


def _edge(gs, gd, feat, selp, selpd, attf):
    e = gs.shape[0]
    cd = gd.shape[1]
    grid = (e // EB_,)
    return pl.pallas_call(
        _edge_body,
        grid=grid,
        in_specs=[
            pl.BlockSpec((EB_, CAT_), lambda i: (i, 0)),
            pl.BlockSpec((EB_, cd), lambda i: (i, 0)),
            pl.BlockSpec((EB_, 1), lambda i: (i, 0)),
            pl.BlockSpec((CAT_, D_), lambda i: (0, 0)),
            pl.BlockSpec((cd, D_), lambda i: (0, 0)),
            pl.BlockSpec((1, D_), lambda i: (0, 0)),
            pl.BlockSpec((CAT_, D_), lambda i: (0, 0)),
            pl.BlockSpec((D_, D_), lambda i: (0, 0)),
            pl.BlockSpec((D_, D_), lambda i: (0, 0)),
        ],
        out_specs=pl.BlockSpec((EB_, CAT_), lambda i: (i, 0)),
        out_shape=jax.ShapeDtypeStruct((e, CAT_), F32),
    )(gs, gd, feat, selp, selpd, attf, _sel_z(), _spread_heads(), _keep4())


def _final2_body(a1_ref, a2_ref, lin_ref, selz_ref, seld_ref, out_ref):
    selz = selz_ref[...]
    seld = seld_ref[...]
    a1 = a1_ref[...]
    a2 = a2_ref[...]
    n1 = jnp.dot(a1, selz, preferred_element_type=F32) / (
        jnp.dot(a1, seld, preferred_element_type=F32) + EPS_)
    n2 = jnp.dot(a2, selz, preferred_element_type=F32) / (
        jnp.dot(a2, seld, preferred_element_type=F32) + EPS_)
    z = jnp.dot(lin_ref[...], selz, preferred_element_type=F32)
    x = n1 + n2 + z
    out_ref[...] = jnp.where(x > 0, x, jnp.exp(jnp.minimum(x, 0.0)) - 1.0)


def _final1_body(a1_ref, lin_ref, selz_ref, seld_ref, out_ref):
    selz = selz_ref[...]
    seld = seld_ref[...]
    a1 = a1_ref[...]
    n1 = jnp.dot(a1, selz, preferred_element_type=F32) / (
        jnp.dot(a1, seld, preferred_element_type=F32) + EPS_)
    z = jnp.dot(lin_ref[...], selz, preferred_element_type=F32)
    x = n1 + z
    out_ref[...] = jnp.where(x > 0, x, jnp.exp(jnp.minimum(x, 0.0)) - 1.0)


def _final(aggs, lin_cat):
    n = lin_cat.shape[0]
    grid = (n // NB_,)
    body = _final2_body if len(aggs) == 2 else _final1_body
    num_big = len(aggs) + 1
    return pl.pallas_call(
        body,
        grid=grid,
        in_specs=(
            [pl.BlockSpec((NB_, CAT_), lambda i: (i, 0)) for _ in range(num_big)]
            + [pl.BlockSpec((CAT_, D_), lambda i: (0, 0)),
               pl.BlockSpec((CAT_, D_), lambda i: (0, 0))]
        ),
        out_specs=pl.BlockSpec((NB_, D_), lambda i: (i, 0)),
        out_shape=jax.ShapeDtypeStruct((n, D_), F32),
    )(*aggs, lin_cat, _sel_z(), _sel_denom())


def _att_matrix(att_list):
    a = jnp.zeros((D_, D_), dtype=F32)
    for slot, (att, col0) in enumerate(att_list):
        vals = att[:, col0:col0 + DK_].reshape(-1)
        rows = jnp.arange(D_)
        cols = H_ * slot + rows // DK_
        a = a.at[rows, cols].add(vals)
    return a


def _attf_row(att):
    row = jnp.zeros((D_,), dtype=F32).at[jnp.arange(H_)].set(att[:, 2 * DK_])
    return row.reshape(1, D_)


def kernel(h_op, h_mac, seq_src, seq_dst, seq_feat, op_mac_src, op_mac_dst,
           op_mac_feat, mac_op_src, mac_op_dst, mac_op_feat, W_op, b_op,
           W_mac, b_mac, att_seq, att_op_mac, att_mac_op):
    n_op = h_op.shape[0]
    n_mac = h_mac.shape[0]

    a_op = _att_matrix([(att_seq, 0), (att_seq, DK_), (att_op_mac, 0),
                        (att_mac_op, DK_)])
    a_mac = _att_matrix([(att_op_mac, DK_), (att_mac_op, 0)])

    t_op = _linear(h_op, W_op.T, b_op.reshape(1, D_), a_op)
    t_mac = _linear(h_mac, W_mac.T, b_mac.reshape(1, D_), a_mac)

    p_op = t_op[:, D_:]
    p_mac = t_mac[:, D_:]

    def relation(t_src, p_dst, src, dst, feat, off_s, off_d, att, n_dst):
        gs = _gather(t_src, src)
        gd = _gather(p_dst, dst)
        cat = _edge(gs, gd, feat, _sel_cols(D_ + off_s), _sel_cols(off_d, D_),
                    _attf_row(att))
        return _scatter(cat, dst, n_dst)

    agg_seq = relation(t_op, p_op, seq_src, seq_dst, seq_feat, 0, 4,
                       att_seq, n_op)
    agg_opmac = relation(t_op, p_mac, op_mac_src, op_mac_dst, op_mac_feat,
                         8, 0, att_op_mac, n_mac)
    agg_macop = relation(t_mac, p_op, mac_op_src, mac_op_dst, mac_op_feat,
                         4, 12, att_mac_op, n_op)

    res_op = _final([agg_seq, agg_macop], t_op)
    res_mac = _final([agg_opmac], t_mac)
    return (res_op, res_mac)


# speedup vs baseline: 1.7817x; 1.0917x over previous
import functools

import jax
import jax.numpy as jnp
from jax import lax
from jax.experimental import pallas as pl

EPS_ = 1e-06
H_ = 4
DK_ = 32
D_ = 128
CAT_ = 256

EB_ = 6000
NT_ = 1000
NB_ = 1000

F32 = jnp.float32


def _sel_cols(off, in_dim=CAT_):
    r = jnp.arange(in_dim)[:, None]
    c = jnp.arange(D_)[None, :]
    return ((r - off == c) & (c < H_)).astype(F32)


def _sel_z():
    r = jnp.arange(CAT_)[:, None]
    c = jnp.arange(D_)[None, :]
    return (r == c).astype(F32)


def _sel_denom():
    r = jnp.arange(CAT_)[:, None]
    c = jnp.arange(D_)[None, :]
    return (r - D_ == c // DK_).astype(F32)


def _spread_heads():
    r = jnp.arange(D_)[:, None]
    c = jnp.arange(D_)[None, :]
    return (r == c // DK_).astype(F32)


def _keep4():
    r = jnp.arange(D_)[:, None]
    c = jnp.arange(D_)[None, :]
    return ((r == c) & (c < H_)).astype(F32)


def _linear_body(h_ref, wt_ref, b_ref, a_ref, out_ref):
    lin = jnp.dot(h_ref[...], wt_ref[...], preferred_element_type=F32) + b_ref[...]
    proj = jnp.dot(lin, a_ref[...], preferred_element_type=F32)
    out_ref[...] = jnp.concatenate([lin, proj], axis=1)


def _linear(h, wt, b, a):
    n = h.shape[0]
    grid = (n // NB_,)
    return pl.pallas_call(
        _linear_body,
        grid=grid,
        in_specs=[
            pl.BlockSpec((NB_, D_), lambda i: (i, 0)),
            pl.BlockSpec((D_, D_), lambda i: (0, 0)),
            pl.BlockSpec((1, D_), lambda i: (0, 0)),
            pl.BlockSpec((D_, D_), lambda i: (0, 0)),
        ],
        out_specs=pl.BlockSpec((NB_, CAT_), lambda i: (i, 0)),
        out_shape=jax.ShapeDtypeStruct((n, CAT_), F32),
    )(h, wt, b, a)


def _gather_body(idx_ref, tab_ref, out_ref):
    j = pl.program_id(1)

    @pl.when(j == 0)
    def _():
        out_ref[...] = jnp.zeros_like(out_ref)

    ids = idx_ref[0]
    rowid = lax.broadcasted_iota(jnp.int32, (NT_, 1), 0) + j * NT_
    oh = (rowid == ids).astype(jnp.bfloat16)
    out_ref[...] += lax.dot_general(
        oh, tab_ref[...].astype(jnp.bfloat16), (((0,), (0,)), ((), ())),
        preferred_element_type=F32)


def _gather(tab, idx):
    e = idx.shape[0]
    n, c = tab.shape
    idx3 = idx.reshape(e // EB_, 1, EB_)
    grid = (e // EB_, n // NT_)
    return pl.pallas_call(
        _gather_body,
        grid=grid,
        in_specs=[
            pl.BlockSpec((1, 1, EB_), lambda i, j: (i, 0, 0)),
            pl.BlockSpec((NT_, c), lambda i, j: (j, 0)),
        ],
        out_specs=pl.BlockSpec((EB_, c), lambda i, j: (i, 0)),
        out_shape=jax.ShapeDtypeStruct((e, c), F32),
    )(idx3, tab)


def _scatter_body(idx_ref, val_ref, out_ref):
    i = pl.program_id(0)
    j = pl.program_id(1)

    @pl.when(j == 0)
    def _():
        out_ref[...] = jnp.zeros_like(out_ref)

    ids = idx_ref[0]
    rowid = lax.broadcasted_iota(jnp.int32, (NT_, 1), 0) + i * NT_
    oh = (rowid == ids).astype(jnp.bfloat16)
    out_ref[...] += jnp.dot(oh, val_ref[...].astype(jnp.bfloat16),
                            preferred_element_type=F32)


def _scatter(vals, idx, n):
    e = idx.shape[0]
    idx3 = idx.reshape(e // EB_, 1, EB_)
    grid = (n // NT_, e // EB_)
    return pl.pallas_call(
        _scatter_body,
        grid=grid,
        in_specs=[
            pl.BlockSpec((1, 1, EB_), lambda i, j: (j, 0, 0)),
            pl.BlockSpec((EB_, CAT_), lambda i, j: (j, 0)),
        ],
        out_specs=pl.BlockSpec((NT_, CAT_), lambda i, j: (i, 0)),
        out_shape=jax.ShapeDtypeStruct((n, CAT_), F32),
    )(idx3, vals)


def _edge_body(gs_ref, gd_ref, feat_ref, selp_ref, selpd_ref, attf_ref,
               selz_ref, sprd_ref, keep_ref, out_ref):
    gs = gs_ref[...]
    gd = gd_ref[...]
    zs = jnp.dot(gs, selz_ref[...], preferred_element_type=F32)
    s = (jnp.dot(gs, selp_ref[...], preferred_element_type=F32)
         + jnp.dot(gd, selpd_ref[...], preferred_element_type=F32)
         + feat_ref[...] * attf_ref[...])
    s = jnp.where(s >= 0, s, 0.2 * s)
    al = jnp.exp(s)
    w = zs * jnp.dot(al, sprd_ref[...], preferred_element_type=F32)
    alk = jnp.dot(al, keep_ref[...], preferred_element_type=F32)
    out_ref[...] = jnp.concatenate([w, alk], axis=1)


def _edge(gs, gd, feat, selp, selpd, attf):
    e = gs.shape[0]
    cd = gd.shape[1]
    grid = (e // EB_,)
    return pl.pallas_call(
        _edge_body,
        grid=grid,
        in_specs=[
            pl.BlockSpec((EB_, CAT_), lambda i: (i, 0)),
            pl.BlockSpec((EB_, cd), lambda i: (i, 0)),
            pl.BlockSpec((EB_, 1), lambda i: (i, 0)),
            pl.BlockSpec((CAT_, D_), lambda i: (0, 0)),
            pl.BlockSpec((cd, D_), lambda i: (0, 0)),
            pl.BlockSpec((1, D_), lambda i: (0, 0)),
            pl.BlockSpec((CAT_, D_), lambda i: (0, 0)),
            pl.BlockSpec((D_, D_), lambda i: (0, 0)),
            pl.BlockSpec((D_, D_), lambda i: (0, 0)),
        ],
        out_specs=pl.BlockSpec((EB_, CAT_), lambda i: (i, 0)),
        out_shape=jax.ShapeDtypeStruct((e, CAT_), F32),
    )(gs, gd, feat, selp, selpd, attf, _sel_z(), _spread_heads(), _keep4())


def _final2_body(a1_ref, a2_ref, lin_ref, selz_ref, seld_ref, out_ref):
    selz = selz_ref[...]
    seld = seld_ref[...]
    a1 = a1_ref[...]
    a2 = a2_ref[...]
    n1 = jnp.dot(a1, selz, preferred_element_type=F32) / (
        jnp.dot(a1, seld, preferred_element_type=F32) + EPS_)
    n2 = jnp.dot(a2, selz, preferred_element_type=F32) / (
        jnp.dot(a2, seld, preferred_element_type=F32) + EPS_)
    z = jnp.dot(lin_ref[...], selz, preferred_element_type=F32)
    x = n1 + n2 + z
    out_ref[...] = jnp.where(x > 0, x, jnp.exp(jnp.minimum(x, 0.0)) - 1.0)


def _final1_body(a1_ref, lin_ref, selz_ref, seld_ref, out_ref):
    selz = selz_ref[...]
    seld = seld_ref[...]
    a1 = a1_ref[...]
    n1 = jnp.dot(a1, selz, preferred_element_type=F32) / (
        jnp.dot(a1, seld, preferred_element_type=F32) + EPS_)
    z = jnp.dot(lin_ref[...], selz, preferred_element_type=F32)
    x = n1 + z
    out_ref[...] = jnp.where(x > 0, x, jnp.exp(jnp.minimum(x, 0.0)) - 1.0)


def _final(aggs, lin_cat):
    n = lin_cat.shape[0]
    grid = (n // NB_,)
    body = _final2_body if len(aggs) == 2 else _final1_body
    num_big = len(aggs) + 1
    return pl.pallas_call(
        body,
        grid=grid,
        in_specs=(
            [pl.BlockSpec((NB_, CAT_), lambda i: (i, 0)) for _ in range(num_big)]
            + [pl.BlockSpec((CAT_, D_), lambda i: (0, 0)),
               pl.BlockSpec((CAT_, D_), lambda i: (0, 0))]
        ),
        out_specs=pl.BlockSpec((NB_, D_), lambda i: (i, 0)),
        out_shape=jax.ShapeDtypeStruct((n, D_), F32),
    )(*aggs, lin_cat, _sel_z(), _sel_denom())


def _att_matrix(att_list):
    a = jnp.zeros((D_, D_), dtype=F32)
    for slot, (att, col0) in enumerate(att_list):
        vals = att[:, col0:col0 + DK_].reshape(-1)
        rows = jnp.arange(D_)
        cols = H_ * slot + rows // DK_
        a = a.at[rows, cols].add(vals)
    return a


def _attf_row(att):
    row = jnp.zeros((D_,), dtype=F32).at[jnp.arange(H_)].set(att[:, 2 * DK_])
    return row.reshape(1, D_)


def kernel(h_op, h_mac, seq_src, seq_dst, seq_feat, op_mac_src, op_mac_dst,
           op_mac_feat, mac_op_src, mac_op_dst, mac_op_feat, W_op, b_op,
           W_mac, b_mac, att_seq, att_op_mac, att_mac_op):
    n_op = h_op.shape[0]
    n_mac = h_mac.shape[0]

    a_op = _att_matrix([(att_seq, 0), (att_seq, DK_), (att_op_mac, 0),
                        (att_mac_op, DK_)])
    a_mac = _att_matrix([(att_op_mac, DK_), (att_mac_op, 0)])

    t_op = _linear(h_op, W_op.T, b_op.reshape(1, D_), a_op)
    t_mac = _linear(h_mac, W_mac.T, b_mac.reshape(1, D_), a_mac)

    p_op = t_op[:, D_:]
    p_mac = t_mac[:, D_:]

    def relation(t_src, p_dst, src, dst, feat, off_s, off_d, att, n_dst):
        gs = _gather(t_src, src)
        gd = _gather(p_dst, dst)
        cat = _edge(gs, gd, feat, _sel_cols(D_ + off_s), _sel_cols(off_d, D_),
                    _attf_row(att))
        return _scatter(cat, dst, n_dst)

    agg_seq = relation(t_op, p_op, seq_src, seq_dst, seq_feat, 0, 4,
                       att_seq, n_op)
    agg_opmac = relation(t_op, p_mac, op_mac_src, op_mac_dst, op_mac_feat,
                         8, 0, att_op_mac, n_mac)
    agg_macop = relation(t_mac, p_op, mac_op_src, mac_op_dst, mac_op_feat,
                         4, 12, att_mac_op, n_op)

    res_op = _final([agg_seq, agg_macop], t_op)
    res_mac = _final([agg_opmac], t_mac)
    return (res_op, res_mac)
